# baseline (speedup 1.0000x reference)
import functools

import jax
import jax.numpy as jnp
from jax import lax
from jax.experimental import pallas as pl
from jax.experimental.pallas import tpu as pltpu
from jax.experimental.pallas import tpu_sc as plsc

D_MODEL = 768
LANES = 16
NVEC = D_MODEL // LANES


def _rsqrt(x):
    xi = lax.bitcast_convert_type(x, jnp.int32)
    yi = jnp.full((LANES,), 0x5F3759DF, jnp.int32) - (xi >> 1)
    y = lax.bitcast_convert_type(yi, jnp.float32)
    for _ in range(3):
        y = y * (1.5 - 0.5 * x * y * y)
    return y


_GATHER_DNUMS = lax.GatherDimensionNumbers(
    offset_dims=(), collapsed_slice_dims=(0,), start_index_map=(0,))


def _permute(v, idx):
    return lax.gather(v, idx[:, None], _GATHER_DNUMS, slice_sizes=(1,),
                      mode=lax.GatherScatterMode.PROMISE_IN_BOUNDS)


def _lane_sum(v):
    for s in (8, 4, 2, 1):
        idx = lax.iota(jnp.int32, LANES) ^ s
        v = v + _permute(v, idx)
    return v


def _make_sc_kernel(n_tokens, n_workers, chunk):
    tpw = n_tokens // n_workers
    n_chunks = tpw // chunk
    mesh = plsc.VectorSubcoreMesh(core_axis_name="c", subcore_axis_name="s")

    @functools.partial(
        pl.kernel,
        mesh=mesh,
        out_type=jax.ShapeDtypeStruct((n_tokens, D_MODEL), jnp.float32),
        scratch_types=[
            pltpu.VMEM((n_chunks, chunk), jnp.int32),
            pltpu.VMEM((chunk, D_MODEL), jnp.float32),
            pltpu.VMEM((chunk, D_MODEL), jnp.float32),
            pltpu.VMEM((D_MODEL,), jnp.float32),
            pltpu.VMEM((D_MODEL,), jnp.float32),
            pltpu.SemaphoreType.DMA,
        ],
    )
    def k(ids_hbm, w_hbm, pos_hbm, gamma_hbm, beta_hbm, out_hbm,
          idx_v, rows_v, pos_v, g_v, b_v, sem):
        nc = 2
        wid = lax.axis_index("s") * nc + lax.axis_index("c")
        base = wid * tpw
        pltpu.sync_copy(ids_hbm.at[wid], idx_v)
        pltpu.sync_copy(gamma_hbm, g_v)
        pltpu.sync_copy(beta_hbm, b_v)

        def row_body(r, _):
            acc = jnp.zeros((LANES,), jnp.float32)
            acc2 = jnp.zeros((LANES,), jnp.float32)
            for j in range(NVEC):
                sl = pl.ds(j * LANES, LANES)
                e = rows_v[r, sl] + pos_v[r, sl]
                rows_v[r, sl] = e
                acc = acc + e
                acc2 = acc2 + e * e
            mean = _lane_sum(acc) * (1.0 / D_MODEL)
            m2 = _lane_sum(acc2) * (1.0 / D_MODEL)
            var = jnp.maximum(m2 - mean * mean, 0.0)
            inv = _rsqrt(var + 1e-12)
            for j in range(NVEC):
                sl = pl.ds(j * LANES, LANES)
                e = rows_v[r, sl]
                rows_v[r, sl] = (e - mean) * inv * g_v[sl] + b_v[sl]
            return 0

        for c in range(n_chunks):
            tok = base + c * chunk
            pbase = tok % pos_hbm.shape[0]
            pltpu.sync_copy(pos_hbm.at[pl.ds(pbase, chunk)], pos_v)
            pltpu.async_copy(w_hbm.at[idx_v.at[c]], rows_v, sem).wait()
            lax.fori_loop(0, chunk, row_body, 0)
            pltpu.sync_copy(rows_v, out_hbm.at[pl.ds(tok, chunk)])

    return k


@jax.jit
def kernel(input_ids, W, pos, gamma, beta):
    batch, seq = input_ids.shape
    n_tokens = batch * seq
    n_workers = 32
    chunk = 64
    ids = input_ids.reshape(n_workers, n_tokens // n_workers // chunk, chunk)
    ids = ids.astype(jnp.int32)
    sc = _make_sc_kernel(n_tokens, n_workers, chunk)
    out = sc(ids, W, pos[0, :seq], gamma, beta)
    return out.reshape(batch, seq, D_MODEL)


# speedup vs baseline: 1.5565x; 1.5565x over previous
import functools

import jax
import jax.numpy as jnp
from jax import lax
from jax.experimental import pallas as pl
from jax.experimental.pallas import tpu as pltpu
from jax.experimental.pallas import tpu_sc as plsc

D_MODEL = 768
LANES = 16
NVEC = D_MODEL // LANES


def _rsqrt(x):
    xi = lax.bitcast_convert_type(x, jnp.int32)
    yi = jnp.full((LANES,), 0x5F3759DF, jnp.int32) - (xi >> 1)
    y = lax.bitcast_convert_type(yi, jnp.float32)
    for _ in range(3):
        y = y * (1.5 - 0.5 * x * y * y)
    return y


_GATHER_DNUMS = lax.GatherDimensionNumbers(
    offset_dims=(), collapsed_slice_dims=(0,), start_index_map=(0,))


def _permute(v, idx):
    return lax.gather(v, idx[:, None], _GATHER_DNUMS, slice_sizes=(1,),
                      mode=lax.GatherScatterMode.PROMISE_IN_BOUNDS)


def _lane_sum(v):
    for s in (8, 4, 2, 1):
        idx = lax.iota(jnp.int32, LANES) ^ s
        v = v + _permute(v, idx)
    return v


def _make_sc_kernel(n_tokens, n_workers, chunk):
    tpw = n_tokens // n_workers
    n_chunks = tpw // chunk
    mesh = plsc.VectorSubcoreMesh(core_axis_name="c", subcore_axis_name="s")

    @functools.partial(
        pl.kernel,
        mesh=mesh,
        out_type=jax.ShapeDtypeStruct((n_tokens, D_MODEL), jnp.float32),
        scratch_types=[
            pltpu.VMEM((n_chunks, chunk), jnp.int32),
            pltpu.VMEM((chunk, D_MODEL), jnp.float32),
            pltpu.VMEM((chunk, D_MODEL), jnp.float32),
            pltpu.VMEM((D_MODEL,), jnp.float32),
            pltpu.VMEM((D_MODEL,), jnp.float32),
            pltpu.VMEM((chunk, LANES), jnp.float32),
            pltpu.VMEM((chunk, LANES), jnp.float32),
            pltpu.SemaphoreType.DMA,
        ],
    )
    def k(ids_hbm, w_hbm, pos_hbm, gamma_hbm, beta_hbm, out_hbm,
          idx_v, rows_v, pos_v, g_v, b_v, mean_v, inv_v, sem):
        nc = 2
        wid = lax.axis_index("s") * nc + lax.axis_index("c")
        base = wid * tpw
        pltpu.sync_copy(ids_hbm.at[wid], idx_v)
        pltpu.sync_copy(gamma_hbm, g_v)
        pltpu.sync_copy(beta_hbm, b_v)

        def stats_body(r, _):
            acc = jnp.zeros((LANES,), jnp.float32)
            acc2 = jnp.zeros((LANES,), jnp.float32)
            for j in range(NVEC):
                sl = pl.ds(j * LANES, LANES)
                e = rows_v[r, sl] + pos_v[r, sl]
                rows_v[r, sl] = e
                acc = acc + e
                acc2 = acc2 + e * e
            mean = _lane_sum(acc) * (1.0 / D_MODEL)
            m2 = _lane_sum(acc2) * (1.0 / D_MODEL)
            var = jnp.maximum(m2 - mean * mean, 0.0)
            mean_v[r, :] = mean
            inv_v[r, :] = _rsqrt(var + 1e-12)
            return 0

        jhalf = NVEC // 2

        def make_norm_body(jb, gregs, bregs):
            def norm_body(r, _):
                mean = mean_v[r, :]
                inv = inv_v[r, :]
                for j in range(jhalf):
                    sl = pl.ds((jb * jhalf + j) * LANES, LANES)
                    e = rows_v[r, sl]
                    rows_v[r, sl] = (e - mean) * inv * gregs[j] + bregs[j]
                return 0
            return norm_body

        for c in range(n_chunks):
            tok = base + c * chunk
            pbase = tok % pos_hbm.shape[0]
            pltpu.sync_copy(pos_hbm.at[pl.ds(pbase, chunk)], pos_v)
            pltpu.async_copy(w_hbm.at[idx_v.at[c]], rows_v, sem).wait()
            lax.fori_loop(0, chunk, stats_body, 0)
            for jb in range(2):
                gregs = [g_v[pl.ds((jb * jhalf + j) * LANES, LANES)]
                         for j in range(jhalf)]
                bregs = [b_v[pl.ds((jb * jhalf + j) * LANES, LANES)]
                         for j in range(jhalf)]
                lax.fori_loop(0, chunk, make_norm_body(jb, gregs, bregs), 0)
            pltpu.sync_copy(rows_v, out_hbm.at[pl.ds(tok, chunk)])

    return k


@jax.jit
def kernel(input_ids, W, pos, gamma, beta):
    batch, seq = input_ids.shape
    n_tokens = batch * seq
    n_workers = 32
    chunk = 64
    ids = input_ids.reshape(n_workers, n_tokens // n_workers // chunk, chunk)
    ids = ids.astype(jnp.int32)
    sc = _make_sc_kernel(n_tokens, n_workers, chunk)
    out = sc(ids, W, pos[0, :seq], gamma, beta)
    return out.reshape(batch, seq, D_MODEL)
